# Initial kernel scaffold; baseline (speedup 1.0000x reference)
import jax
import jax.numpy as jnp
from jax.experimental import pallas as pl


def kernel(original_pts, query_pts, W_in, b_in, W_b0, b_b0, W_b1, b_b1, W_b2, b_b2, R1, Rb1, R2, Rb2, R3, Rb3):
    raise NotImplementedError("write your pallas kernel here")


# speedup vs baseline: 23.9231x; 23.9231x over previous
import functools

import jax
import jax.numpy as jnp
from jax import lax
from jax.experimental import pallas as pl
from jax.experimental.pallas import tpu as pltpu
from jax.experimental.pallas import tpu_sc as plsc

_B, _N, _M, _C = 2, 4096, 4096, 128
_K = 3
_TN = 512
_TM = 256
_BIG = 1e30
_CAT = 4 * _C
_ROWS = _B * _K * _M
_CH = 64


def _dot(a, b):
    return jax.lax.dot_general(a, b, (((1,), (0,)), ((), ())),
                               preferred_element_type=jnp.float32)


def _full(shape):
    return pl.BlockSpec(shape, lambda *_: (0,) * len(shape))


def _feat_body(ptsT_ref, W_in_ref, b_in_ref, W0_ref, b0_ref, W1_ref, b1_ref,
               W2_ref, b2_ref, cat_ref, g_ref):
    pts = ptsT_ref[0]
    f0 = jnp.maximum(_dot(pts, W_in_ref[...]) + b_in_ref[...], 0.0)
    f1 = jnp.maximum(_dot(f0, W0_ref[...]) + b0_ref[...], 0.0)
    f2 = jnp.maximum(_dot(f1, W1_ref[...]) + b1_ref[...], 0.0)
    f3 = jnp.maximum(_dot(f2, W2_ref[...]) + b2_ref[...], 0.0)
    cat_ref[0] = jnp.concatenate([f0, f1, f2, f3], axis=1)
    gmax = jnp.max(f3, axis=0, keepdims=True)

    ni = pl.program_id(1)

    @pl.when(ni == 0)
    def _():
        g_ref[0] = gmax

    @pl.when(ni > 0)
    def _():
        g_ref[0] = jnp.maximum(g_ref[0], gmax)


def _knn_body(qT_ref, pts_ref, idx_ref, w_ref):
    qT = qT_ref[0]
    pts = pts_ref[0]
    bi = pl.program_id(0)

    p2 = jnp.sum(pts * pts, axis=0, keepdims=True)
    q2 = jnp.sum(qT * qT, axis=1, keepdims=True)
    inner = _dot(qT, pts)
    d2 = jnp.maximum(q2 + p2 - 2.0 * inner, 0.0)

    iota = lax.broadcasted_iota(jnp.int32, (_TM, _N), 1)
    px = pts[0:1, :]
    py = pts[1:2, :]
    pz = pts[2:3, :]
    qx = qT[:, 0:1]
    qy = qT[:, 1:2]
    qz = qT[:, 2:3]

    d2w = d2
    recips = []
    sels = []
    for _ in range(_K):
        mval = jnp.min(d2w, axis=1, keepdims=True)
        midx = jnp.min(jnp.where(d2w == mval, iota, _N),
                       axis=1, keepdims=True)
        sel = iota == midx
        d2w = jnp.where(sel, _BIG, d2w)
        nx = jnp.sum(jnp.where(sel, px, 0.0), axis=1, keepdims=True)
        ny = jnp.sum(jnp.where(sel, py, 0.0), axis=1, keepdims=True)
        nz = jnp.sum(jnp.where(sel, pz, 0.0), axis=1, keepdims=True)
        dx = nx - qx
        dy = ny - qy
        dz = nz - qz
        de = (dx * dx + dy * dy) + dz * dz
        dist = jnp.sqrt(de + 1e-12)
        recips.append(1.0 / (dist + 1e-8))
        sels.append(sel)
        idx_ref[0, len(sels) - 1] = (midx + bi * _N).astype(jnp.int32)

    norm = (recips[0] + recips[1]) + recips[2]
    for k in range(_K):
        w_ref[0, k] = recips[k] / norm


def _make_sc_gather():
    info = plsc.get_sparse_core_info()
    nw = info.num_cores * info.num_subcores
    rows_per_w = _ROWS // nw
    n_chunks = rows_per_w // _CH
    mesh = plsc.VectorSubcoreMesh(core_axis_name="c", subcore_axis_name="s")

    @functools.partial(
        pl.kernel, mesh=mesh,
        out_type=jax.ShapeDtypeStruct((_ROWS, _CAT), jnp.float32),
        scratch_types=[
            pltpu.VMEM((_CH,), jnp.int32),
            pltpu.VMEM((_CH, _CAT), jnp.float32),
            pltpu.SemaphoreType.DMA,
        ],
    )
    def gather(table_hbm, idx_hbm, out_hbm, idx_v, rows_v, sem):
        wid = lax.axis_index("s") * info.num_cores + lax.axis_index("c")
        base = wid * rows_per_w
        for j in range(n_chunks):
            off = base + j * _CH
            pltpu.sync_copy(idx_hbm.at[pl.ds(off, _CH)], idx_v)
            pltpu.async_copy(table_hbm.at[idx_v], rows_v, sem).wait()
            pltpu.sync_copy(rows_v, out_hbm.at[pl.ds(off, _CH)])

    return gather


def _reg_body(qT_ref, G_ref, w_ref, g_ref, R1_ref, Rb1_ref, R2_ref, Rb2_ref,
              R3_ref, Rb3_ref, out_ref):
    qT = qT_ref[0]
    acc = w_ref[0, 0] * G_ref[0, 0]
    acc = acc + w_ref[0, 1] * G_ref[0, 1]
    acc = acc + w_ref[0, 2] * G_ref[0, 2]
    gb = jnp.broadcast_to(g_ref[0], (_TM, _C))
    aggT = jnp.concatenate([qT, acc, gb], axis=1)
    h1 = jnp.maximum(_dot(aggT, R1_ref[...]) + Rb1_ref[...], 0.0)
    h2 = jnp.maximum(_dot(h1, R2_ref[...]) + Rb2_ref[...], 0.0)
    out_ref[0] = _dot(h2, R3_ref[...]) + Rb3_ref[...]


@jax.jit
def kernel(original_pts, query_pts, W_in, b_in, W_b0, b_b0, W_b1, b_b1,
           W_b2, b_b2, R1, Rb1, R2, Rb2, R3, Rb3):
    b, _, n = original_pts.shape
    m = query_pts.shape[-1]
    ptsT = jnp.transpose(original_pts, (0, 2, 1))
    qT = jnp.transpose(query_pts, (0, 2, 1))

    cat, g = pl.pallas_call(
        _feat_body,
        grid=(b, n // _TN),
        in_specs=[
            pl.BlockSpec((1, _TN, 3), lambda bi, ni: (bi, ni, 0)),
            _full((3, _C)), _full((1, _C)),
            _full((_C, _C)), _full((1, _C)),
            _full((_C, _C)), _full((1, _C)),
            _full((_C, _C)), _full((1, _C)),
        ],
        out_specs=[
            pl.BlockSpec((1, _TN, _CAT), lambda bi, ni: (bi, ni, 0)),
            pl.BlockSpec((1, 1, _C), lambda bi, ni: (bi, 0, 0)),
        ],
        out_shape=[
            jax.ShapeDtypeStruct((b, n, _CAT), jnp.float32),
            jax.ShapeDtypeStruct((b, 1, _C), jnp.float32),
        ],
    )(ptsT, W_in, b_in.reshape(1, -1), W_b0, b_b0.reshape(1, -1),
      W_b1, b_b1.reshape(1, -1), W_b2, b_b2.reshape(1, -1))

    idx, w = pl.pallas_call(
        _knn_body,
        grid=(b, m // _TM),
        in_specs=[
            pl.BlockSpec((1, _TM, 3), lambda bi, mi: (bi, mi, 0)),
            pl.BlockSpec((1, 3, n), lambda bi, mi: (bi, 0, 0)),
        ],
        out_specs=[
            pl.BlockSpec((1, _K, _TM, 1), lambda bi, mi: (bi, 0, mi, 0)),
            pl.BlockSpec((1, _K, _TM, 1), lambda bi, mi: (bi, 0, mi, 0)),
        ],
        out_shape=[
            jax.ShapeDtypeStruct((b, _K, m, 1), jnp.int32),
            jax.ShapeDtypeStruct((b, _K, m, 1), jnp.float32),
        ],
    )(qT, original_pts)

    table = cat.reshape(b * n, _CAT)
    idx_flat = idx.reshape(_ROWS)
    G = _make_sc_gather()(table, idx_flat)
    G = G.reshape(b, _K, m, _CAT)

    out = pl.pallas_call(
        _reg_body,
        grid=(b, m // _TM),
        in_specs=[
            pl.BlockSpec((1, _TM, 3), lambda bi, mi: (bi, mi, 0)),
            pl.BlockSpec((1, _K, _TM, _CAT), lambda bi, mi: (bi, 0, mi, 0)),
            pl.BlockSpec((1, _K, _TM, 1), lambda bi, mi: (bi, 0, mi, 0)),
            pl.BlockSpec((1, 1, _C), lambda bi, mi: (bi, 0, 0)),
            _full((3 + _CAT + _C, 256)), _full((1, 256)),
            _full((256, 64)), _full((1, 64)),
            _full((64, 1)), _full((1, 1)),
        ],
        out_specs=pl.BlockSpec((1, _TM, 1), lambda bi, mi: (bi, mi, 0)),
        out_shape=jax.ShapeDtypeStruct((b, m, 1), jnp.float32),
    )(qT, G, w, g, R1, Rb1.reshape(1, -1),
      R2, Rb2.reshape(1, -1), R3, Rb3.reshape(1, -1))

    return jnp.transpose(out, (0, 2, 1))


# speedup vs baseline: 34.6316x; 1.4476x over previous
import functools

import jax
import jax.numpy as jnp
from jax import lax
from jax.experimental import pallas as pl
from jax.experimental.pallas import tpu as pltpu
from jax.experimental.pallas import tpu_sc as plsc

_B, _N, _M, _C = 2, 4096, 4096, 128
_K = 3
_TN = 512
_TM = 256
_BIG = 1e30
_CAT = 4 * _C
_ROW = _CAT + _C
_ROWS = _B * _K * _M
_CH = 64


def _dot(a, b):
    return jax.lax.dot_general(a, b, (((1,), (0,)), ((), ())),
                               preferred_element_type=jnp.float32)


def _full(shape):
    return pl.BlockSpec(shape, lambda *_: (0,) * len(shape))


def _feat_body(ptsT_ref, W_in_ref, b_in_ref, W0_ref, b0_ref, W1_ref, b1_ref,
               W2_ref, b2_ref, cat_ref, g_ref):
    pts = ptsT_ref[0]
    f0 = jnp.maximum(_dot(pts, W_in_ref[...]) + b_in_ref[...], 0.0)
    f1 = jnp.maximum(_dot(f0, W0_ref[...]) + b0_ref[...], 0.0)
    f2 = jnp.maximum(_dot(f1, W1_ref[...]) + b1_ref[...], 0.0)
    f3 = jnp.maximum(_dot(f2, W2_ref[...]) + b2_ref[...], 0.0)
    pad = jnp.zeros((_TN, _ROW - _CAT - 3), jnp.float32)
    cat_ref[0] = jnp.concatenate([f0, f1, f2, f3, pts, pad], axis=1)
    gmax = jnp.max(f3, axis=0, keepdims=True)

    ni = pl.program_id(1)

    @pl.when(ni == 0)
    def _():
        g_ref[0] = gmax

    @pl.when(ni > 0)
    def _():
        g_ref[0] = jnp.maximum(g_ref[0], gmax)


def _knn_body(qT_ref, pts_ref, idx_ref):
    qT = qT_ref[0]
    pts = pts_ref[0]
    bi = pl.program_id(0)

    p2 = jnp.sum(pts * pts, axis=0, keepdims=True)
    q2 = jnp.sum(qT * qT, axis=1, keepdims=True)
    inner = _dot(qT, pts)
    d2 = jnp.maximum(q2 + p2 - 2.0 * inner, 0.0)

    iota = lax.broadcasted_iota(jnp.int32, (_TM, _N), 1)
    d2w = d2
    for k in range(_K):
        mval = jnp.min(d2w, axis=1, keepdims=True)
        midx = jnp.min(jnp.where(d2w == mval, iota, _N),
                       axis=1, keepdims=True)
        idx_ref[0, k] = midx + bi * _N
        if k + 1 < _K:
            d2w = jnp.where(iota == midx, _BIG, d2w)


def _make_sc_gather():
    info = plsc.get_sparse_core_info()
    nw = info.num_cores * info.num_subcores
    rows_per_w = _ROWS // nw
    n_chunks = rows_per_w // _CH
    mesh = plsc.VectorSubcoreMesh(core_axis_name="c", subcore_axis_name="s")

    @functools.partial(
        pl.kernel, mesh=mesh,
        out_type=jax.ShapeDtypeStruct((_ROWS, _ROW), jnp.float32),
        scratch_types=[
            pltpu.VMEM((_CH,), jnp.int32),
            pltpu.VMEM((_CH, _ROW), jnp.float32),
            pltpu.SemaphoreType.DMA,
        ],
    )
    def gather(table_hbm, idx_hbm, out_hbm, idx_v, rows_v, sem):
        wid = lax.axis_index("s") * info.num_cores + lax.axis_index("c")
        base = wid * rows_per_w
        for j in range(n_chunks):
            off = base + j * _CH
            pltpu.sync_copy(idx_hbm.at[pl.ds(off, _CH)], idx_v)
            pltpu.async_copy(table_hbm.at[idx_v], rows_v, sem).wait()
            pltpu.sync_copy(rows_v, out_hbm.at[pl.ds(off, _CH)])

    return gather


def _reg_body(qT_ref, G_ref, g_ref, R1_ref, Rb1_ref, R2_ref, Rb2_ref,
              R3_ref, Rb3_ref, out_ref):
    qT = qT_ref[0]
    qx = qT[:, 0:1]
    qy = qT[:, 1:2]
    qz = qT[:, 2:3]
    recips = []
    for k in range(_K):
        dx = G_ref[0, k][:, _CAT:_CAT + 1] - qx
        dy = G_ref[0, k][:, _CAT + 1:_CAT + 2] - qy
        dz = G_ref[0, k][:, _CAT + 2:_CAT + 3] - qz
        de = (dx * dx + dy * dy) + dz * dz
        dist = jnp.sqrt(de + 1e-12)
        recips.append(1.0 / (dist + 1e-8))
    norm = (recips[0] + recips[1]) + recips[2]
    acc = (recips[0] / norm) * G_ref[0, 0][:, :_CAT]
    acc = acc + (recips[1] / norm) * G_ref[0, 1][:, :_CAT]
    acc = acc + (recips[2] / norm) * G_ref[0, 2][:, :_CAT]
    gb = jnp.broadcast_to(g_ref[0], (_TM, _C))
    aggT = jnp.concatenate([qT, acc, gb], axis=1)
    h1 = jnp.maximum(_dot(aggT, R1_ref[...]) + Rb1_ref[...], 0.0)
    h2 = jnp.maximum(_dot(h1, R2_ref[...]) + Rb2_ref[...], 0.0)
    out_ref[0] = _dot(h2, R3_ref[...]) + Rb3_ref[...]


@jax.jit
def kernel(original_pts, query_pts, W_in, b_in, W_b0, b_b0, W_b1, b_b1,
           W_b2, b_b2, R1, Rb1, R2, Rb2, R3, Rb3):
    b, _, n = original_pts.shape
    m = query_pts.shape[-1]
    ptsT = jnp.transpose(original_pts, (0, 2, 1))
    qT = jnp.transpose(query_pts, (0, 2, 1))

    cat, g = pl.pallas_call(
        _feat_body,
        grid=(b, n // _TN),
        in_specs=[
            pl.BlockSpec((1, _TN, 3), lambda bi, ni: (bi, ni, 0)),
            _full((3, _C)), _full((1, _C)),
            _full((_C, _C)), _full((1, _C)),
            _full((_C, _C)), _full((1, _C)),
            _full((_C, _C)), _full((1, _C)),
        ],
        out_specs=[
            pl.BlockSpec((1, _TN, _ROW), lambda bi, ni: (bi, ni, 0)),
            pl.BlockSpec((1, 1, _C), lambda bi, ni: (bi, 0, 0)),
        ],
        out_shape=[
            jax.ShapeDtypeStruct((b, n, _ROW), jnp.float32),
            jax.ShapeDtypeStruct((b, 1, _C), jnp.float32),
        ],
    )(ptsT, W_in, b_in.reshape(1, -1), W_b0, b_b0.reshape(1, -1),
      W_b1, b_b1.reshape(1, -1), W_b2, b_b2.reshape(1, -1))

    idx = pl.pallas_call(
        _knn_body,
        grid=(b, m // _TM),
        in_specs=[
            pl.BlockSpec((1, _TM, 3), lambda bi, mi: (bi, mi, 0)),
            pl.BlockSpec((1, 3, n), lambda bi, mi: (bi, 0, 0)),
        ],
        out_specs=pl.BlockSpec((1, _K, _TM, 1), lambda bi, mi: (bi, 0, mi, 0)),
        out_shape=jax.ShapeDtypeStruct((b, _K, m, 1), jnp.int32),
    )(qT, original_pts)

    table = cat.reshape(b * n, _ROW)
    idx_flat = idx.reshape(_ROWS)
    G = _make_sc_gather()(table, idx_flat)
    G = G.reshape(b, _K, m, _ROW)

    out = pl.pallas_call(
        _reg_body,
        grid=(b, m // _TM),
        in_specs=[
            pl.BlockSpec((1, _TM, 3), lambda bi, mi: (bi, mi, 0)),
            pl.BlockSpec((1, _K, _TM, _ROW), lambda bi, mi: (bi, 0, mi, 0)),
            pl.BlockSpec((1, 1, _C), lambda bi, mi: (bi, 0, 0)),
            _full((3 + _CAT + _C, 256)), _full((1, 256)),
            _full((256, 64)), _full((1, 64)),
            _full((64, 1)), _full((1, 1)),
        ],
        out_specs=pl.BlockSpec((1, _TM, 1), lambda bi, mi: (bi, mi, 0)),
        out_shape=jax.ShapeDtypeStruct((b, m, 1), jnp.float32),
    )(qT, G, g, R1, Rb1.reshape(1, -1),
      R2, Rb2.reshape(1, -1), R3, Rb3.reshape(1, -1))

    return jnp.transpose(out, (0, 2, 1))


# speedup vs baseline: 39.7104x; 1.1467x over previous
import functools

import jax
import jax.numpy as jnp
from jax import lax
from jax.experimental import pallas as pl
from jax.experimental.pallas import tpu as pltpu
from jax.experimental.pallas import tpu_sc as plsc

_B, _N, _M, _C = 2, 4096, 4096, 128
_K = 3
_TN = 512
_TM = 256
_BIG = 1e30
_CAT = 4 * _C
_ROW = _CAT + _C
_ROWS_B = _K * _M
_CH = 96


def _dot(a, b):
    return jax.lax.dot_general(a, b, (((1,), (0,)), ((), ())),
                               preferred_element_type=jnp.float32)


def _full(shape):
    return pl.BlockSpec(shape, lambda *_: (0,) * len(shape))


def _feat_body(ptsT_ref, W_in_ref, b_in_ref, W0_ref, b0_ref, W1_ref, b1_ref,
               W2_ref, b2_ref, cat_ref, g_ref):
    pts = ptsT_ref[0]
    f0 = jnp.maximum(_dot(pts, W_in_ref[...]) + b_in_ref[...], 0.0)
    f1 = jnp.maximum(_dot(f0, W0_ref[...]) + b0_ref[...], 0.0)
    f2 = jnp.maximum(_dot(f1, W1_ref[...]) + b1_ref[...], 0.0)
    f3 = jnp.maximum(_dot(f2, W2_ref[...]) + b2_ref[...], 0.0)
    pad = jnp.zeros((_TN, _ROW - _CAT - 3), jnp.float32)
    cat_ref[0] = jnp.concatenate([f0, f1, f2, f3, pts, pad], axis=1)
    gmax = jnp.max(f3, axis=0, keepdims=True)

    ni = pl.program_id(1)

    @pl.when(ni == 0)
    def _():
        g_ref[0] = gmax

    @pl.when(ni > 0)
    def _():
        g_ref[0] = jnp.maximum(g_ref[0], gmax)


def _knn_body(qT_ref, pts_ref, idx_ref, *, row_offset):
    qT = qT_ref[0]
    pts = pts_ref[0]

    p2 = jnp.sum(pts * pts, axis=0, keepdims=True)
    q2 = jnp.sum(qT * qT, axis=1, keepdims=True)
    inner = _dot(qT, pts)
    d2 = jnp.maximum(q2 + p2 - 2.0 * inner, 0.0)

    iota = lax.broadcasted_iota(jnp.int32, (_TM, _N), 1)
    d2w = d2
    for k in range(_K):
        mval = jnp.min(d2w, axis=1, keepdims=True)
        midx = jnp.min(jnp.where(d2w == mval, iota, _N),
                       axis=1, keepdims=True)
        idx_ref[0, k] = midx + row_offset
        if k + 1 < _K:
            d2w = jnp.where(iota == midx, _BIG, d2w)


@functools.cache
def _make_sc_gather():
    info = plsc.get_sparse_core_info()
    nw = info.num_cores * info.num_subcores
    rows_per_w = _ROWS_B // nw
    n_chunks = rows_per_w // _CH
    mesh = plsc.VectorSubcoreMesh(core_axis_name="c", subcore_axis_name="s")

    @functools.partial(
        pl.kernel, mesh=mesh,
        out_type=jax.ShapeDtypeStruct((_ROWS_B, _ROW), jnp.float32),
        scratch_types=[
            pltpu.VMEM((rows_per_w,), jnp.int32),
            pltpu.VMEM((_CH, _ROW), jnp.float32),
            pltpu.VMEM((_CH, _ROW), jnp.float32),
            pltpu.SemaphoreType.DMA,
            pltpu.SemaphoreType.DMA,
            pltpu.SemaphoreType.DMA,
            pltpu.SemaphoreType.DMA,
        ],
    )
    def gather(table_hbm, idx_hbm, out_hbm, idx_v, buf0, buf1,
               gsem0, gsem1, wsem0, wsem1):
        wid = lax.axis_index("s") * info.num_cores + lax.axis_index("c")
        base = wid * rows_per_w
        pltpu.sync_copy(idx_hbm.at[pl.ds(base, rows_per_w)], idx_v)
        bufs = (buf0, buf1)
        gsems = (gsem0, gsem1)
        wsems = (wsem0, wsem1)
        gh = [None, None]
        wh = [None, None]
        for j in range(n_chunks):
            s = j % 2
            if wh[s] is not None:
                wh[s].wait()
                wh[s] = None
            gh[s] = pltpu.async_copy(
                table_hbm.at[idx_v.at[pl.ds(j * _CH, _CH)]], bufs[s],
                gsems[s])
            if gh[1 - s] is not None:
                gh[1 - s].wait()
                gh[1 - s] = None
                wh[1 - s] = pltpu.async_copy(
                    bufs[1 - s],
                    out_hbm.at[pl.ds(base + (j - 1) * _CH, _CH)],
                    wsems[1 - s])
        last = n_chunks - 1
        s = last % 2
        gh[s].wait()
        wh[s] = pltpu.async_copy(
            bufs[s], out_hbm.at[pl.ds(base + last * _CH, _CH)], wsems[s])
        for s in range(2):
            if wh[s] is not None:
                wh[s].wait()

    return gather


def _reg_body(qT_ref, G_ref, g_ref, R1_ref, Rb1_ref, R2_ref, Rb2_ref,
              R3_ref, Rb3_ref, out_ref):
    qT = qT_ref[0]
    qx = qT[:, 0:1]
    qy = qT[:, 1:2]
    qz = qT[:, 2:3]
    recips = []
    for k in range(_K):
        dx = G_ref[k][:, _CAT:_CAT + 1] - qx
        dy = G_ref[k][:, _CAT + 1:_CAT + 2] - qy
        dz = G_ref[k][:, _CAT + 2:_CAT + 3] - qz
        de = (dx * dx + dy * dy) + dz * dz
        dist = jnp.sqrt(de + 1e-12)
        recips.append(1.0 / (dist + 1e-8))
    norm = (recips[0] + recips[1]) + recips[2]
    acc = (recips[0] / norm) * G_ref[0][:, :_CAT]
    acc = acc + (recips[1] / norm) * G_ref[1][:, :_CAT]
    acc = acc + (recips[2] / norm) * G_ref[2][:, :_CAT]
    gb = jnp.broadcast_to(g_ref[0], (_TM, _C))
    aggT = jnp.concatenate([qT, acc, gb], axis=1)
    h1 = jnp.maximum(_dot(aggT, R1_ref[...]) + Rb1_ref[...], 0.0)
    h2 = jnp.maximum(_dot(h1, R2_ref[...]) + Rb2_ref[...], 0.0)
    out_ref[0] = _dot(h2, R3_ref[...]) + Rb3_ref[...]


@jax.jit
def kernel(original_pts, query_pts, W_in, b_in, W_b0, b_b0, W_b1, b_b1,
           W_b2, b_b2, R1, Rb1, R2, Rb2, R3, Rb3):
    b, _, n = original_pts.shape
    m = query_pts.shape[-1]
    ptsT = jnp.transpose(original_pts, (0, 2, 1))
    qT = jnp.transpose(query_pts, (0, 2, 1))

    cat, g = pl.pallas_call(
        _feat_body,
        grid=(b, n // _TN),
        in_specs=[
            pl.BlockSpec((1, _TN, 3), lambda bi, ni: (bi, ni, 0)),
            _full((3, _C)), _full((1, _C)),
            _full((_C, _C)), _full((1, _C)),
            _full((_C, _C)), _full((1, _C)),
            _full((_C, _C)), _full((1, _C)),
        ],
        out_specs=[
            pl.BlockSpec((1, _TN, _ROW), lambda bi, ni: (bi, ni, 0)),
            pl.BlockSpec((1, 1, _C), lambda bi, ni: (bi, 0, 0)),
        ],
        out_shape=[
            jax.ShapeDtypeStruct((b, n, _ROW), jnp.float32),
            jax.ShapeDtypeStruct((b, 1, _C), jnp.float32),
        ],
    )(ptsT, W_in, b_in.reshape(1, -1), W_b0, b_b0.reshape(1, -1),
      W_b1, b_b1.reshape(1, -1), W_b2, b_b2.reshape(1, -1))

    table = cat.reshape(b * n, _ROW)
    sc_gather = _make_sc_gather()

    outs = []
    for bi in range(b):
        idx = pl.pallas_call(
            functools.partial(_knn_body, row_offset=bi * n),
            grid=(m // _TM,),
            in_specs=[
                pl.BlockSpec((1, _TM, 3), lambda mi, bi=bi: (bi, mi, 0)),
                pl.BlockSpec((1, 3, n), lambda mi, bi=bi: (bi, 0, 0)),
            ],
            out_specs=pl.BlockSpec((1, _K, _TM, 1), lambda mi: (0, 0, mi, 0)),
            out_shape=jax.ShapeDtypeStruct((1, _K, m, 1), jnp.int32),
        )(qT, original_pts)

        G = sc_gather(table, idx.reshape(_ROWS_B))
        G = G.reshape(_K, m, _ROW)

        out_b = pl.pallas_call(
            _reg_body,
            grid=(m // _TM,),
            in_specs=[
                pl.BlockSpec((1, _TM, 3), lambda mi, bi=bi: (bi, mi, 0)),
                pl.BlockSpec((_K, _TM, _ROW), lambda mi: (0, mi, 0)),
                pl.BlockSpec((1, 1, _C), lambda mi, bi=bi: (bi, 0, 0)),
                _full((3 + _CAT + _C, 256)), _full((1, 256)),
                _full((256, 64)), _full((1, 64)),
                _full((64, 1)), _full((1, 1)),
            ],
            out_specs=pl.BlockSpec((1, _TM, 1), lambda mi: (0, mi, 0)),
            out_shape=jax.ShapeDtypeStruct((1, m, 1), jnp.float32),
        )(qT, G, g, R1, Rb1.reshape(1, -1),
          R2, Rb2.reshape(1, -1), R3, Rb3.reshape(1, -1))
        outs.append(out_b)

    out = jnp.concatenate(outs, axis=0)
    return jnp.transpose(out, (0, 2, 1))


# speedup vs baseline: 40.3705x; 1.0166x over previous
import functools

import jax
import jax.numpy as jnp
from jax import lax
from jax.experimental import pallas as pl
from jax.experimental.pallas import tpu as pltpu
from jax.experimental.pallas import tpu_sc as plsc

_B, _N, _M, _C = 2, 4096, 4096, 128
_K = 3
_TN = 512
_TM = 256
_BIG = 1e30
_CAT = 4 * _C
_ROW = _CAT + _C
_ROWS_B = _K * _M
_CH = 96


def _dot(a, b):
    return jax.lax.dot_general(a, b, (((1,), (0,)), ((), ())),
                               preferred_element_type=jnp.float32)


def _full(shape):
    return pl.BlockSpec(shape, lambda *_: (0,) * len(shape))


def _feat_body(ptsT_ref, W_in_ref, b_in_ref, W0_ref, b0_ref, W1_ref, b1_ref,
               W2_ref, b2_ref, cat_ref, g_ref):
    pts = ptsT_ref[0]
    f0 = jnp.maximum(_dot(pts, W_in_ref[...]) + b_in_ref[...], 0.0)
    f1 = jnp.maximum(_dot(f0, W0_ref[...]) + b0_ref[...], 0.0)
    f2 = jnp.maximum(_dot(f1, W1_ref[...]) + b1_ref[...], 0.0)
    f3 = jnp.maximum(_dot(f2, W2_ref[...]) + b2_ref[...], 0.0)
    pad = jnp.zeros((_TN, _ROW - _CAT - 3), jnp.float32)
    cat_ref[0] = jnp.concatenate([f0, f1, f2, f3, pts, pad], axis=1)
    gmax = jnp.max(f3, axis=0, keepdims=True)

    ni = pl.program_id(1)

    @pl.when(ni == 0)
    def _():
        g_ref[0] = gmax

    @pl.when(ni > 0)
    def _():
        g_ref[0] = jnp.maximum(g_ref[0], gmax)


def _knn_body(qT_ref, pts_ref, idx_ref, *, row_offset):
    qT = qT_ref[0]
    pts = pts_ref[0]

    p2 = jnp.sum(pts * pts, axis=0, keepdims=True)
    q2 = jnp.sum(qT * qT, axis=1, keepdims=True)
    inner_m2 = _dot(qT, -2.0 * pts)
    d2 = jnp.maximum((q2 + p2) + inner_m2, 0.0)

    iota = lax.broadcasted_iota(jnp.int32, (_TM, _N), 1)
    d2w = d2
    for k in range(_K):
        mval = jnp.min(d2w, axis=1, keepdims=True)
        midx = jnp.min(jnp.where(d2w == mval, iota, _N),
                       axis=1, keepdims=True)
        idx_ref[0, k] = midx + row_offset
        if k + 1 < _K:
            d2w = jnp.where(iota == midx, _BIG, d2w)


@functools.cache
def _make_sc_gather():
    info = plsc.get_sparse_core_info()
    nw = info.num_cores * info.num_subcores
    rows_per_w = _ROWS_B // nw
    n_chunks = rows_per_w // _CH
    mesh = plsc.VectorSubcoreMesh(core_axis_name="c", subcore_axis_name="s")

    @functools.partial(
        pl.kernel, mesh=mesh,
        out_type=jax.ShapeDtypeStruct((_ROWS_B, _ROW), jnp.float32),
        scratch_types=[
            pltpu.VMEM((rows_per_w,), jnp.int32),
            pltpu.VMEM((_CH, _ROW), jnp.float32),
            pltpu.VMEM((_CH, _ROW), jnp.float32),
            pltpu.SemaphoreType.DMA,
            pltpu.SemaphoreType.DMA,
            pltpu.SemaphoreType.DMA,
            pltpu.SemaphoreType.DMA,
        ],
    )
    def gather(table_hbm, idx_hbm, out_hbm, idx_v, buf0, buf1,
               gsem0, gsem1, wsem0, wsem1):
        wid = lax.axis_index("s") * info.num_cores + lax.axis_index("c")
        base = wid * rows_per_w
        pltpu.sync_copy(idx_hbm.at[pl.ds(base, rows_per_w)], idx_v)
        bufs = (buf0, buf1)
        gsems = (gsem0, gsem1)
        wsems = (wsem0, wsem1)
        gh = [None, None]
        wh = [None, None]
        for j in range(n_chunks):
            s = j % 2
            if wh[s] is not None:
                wh[s].wait()
                wh[s] = None
            gh[s] = pltpu.async_copy(
                table_hbm.at[idx_v.at[pl.ds(j * _CH, _CH)]], bufs[s],
                gsems[s])
            if gh[1 - s] is not None:
                gh[1 - s].wait()
                gh[1 - s] = None
                wh[1 - s] = pltpu.async_copy(
                    bufs[1 - s],
                    out_hbm.at[pl.ds(base + (j - 1) * _CH, _CH)],
                    wsems[1 - s])
        last = n_chunks - 1
        s = last % 2
        gh[s].wait()
        wh[s] = pltpu.async_copy(
            bufs[s], out_hbm.at[pl.ds(base + last * _CH, _CH)], wsems[s])
        for s in range(2):
            if wh[s] is not None:
                wh[s].wait()

    return gather


def _reg_body(qT_ref, G_ref, g_ref, R1_ref, Rb1_ref, R2_ref, Rb2_ref,
              R3_ref, Rb3_ref, out_ref):
    qT = qT_ref[0]
    qx = qT[:, 0:1]
    qy = qT[:, 1:2]
    qz = qT[:, 2:3]
    recips = []
    for k in range(_K):
        dx = G_ref[k][:, _CAT:_CAT + 1] - qx
        dy = G_ref[k][:, _CAT + 1:_CAT + 2] - qy
        dz = G_ref[k][:, _CAT + 2:_CAT + 3] - qz
        de = (dx * dx + dy * dy) + dz * dz
        dist = jnp.sqrt(de + 1e-12)
        recips.append(1.0 / (dist + 1e-8))
    norm = (recips[0] + recips[1]) + recips[2]
    acc = (recips[0] / norm) * G_ref[0][:, :_CAT]
    acc = acc + (recips[1] / norm) * G_ref[1][:, :_CAT]
    acc = acc + (recips[2] / norm) * G_ref[2][:, :_CAT]
    gb = jnp.broadcast_to(g_ref[0], (_TM, _C))
    aggT = jnp.concatenate([qT, acc, gb], axis=1)
    h1 = jnp.maximum(_dot(aggT, R1_ref[...]) + Rb1_ref[...], 0.0)
    h2 = jnp.maximum(_dot(h1, R2_ref[...]) + Rb2_ref[...], 0.0)
    out_ref[0] = _dot(h2, R3_ref[...]) + Rb3_ref[...]


@jax.jit
def kernel(original_pts, query_pts, W_in, b_in, W_b0, b_b0, W_b1, b_b1,
           W_b2, b_b2, R1, Rb1, R2, Rb2, R3, Rb3):
    b, _, n = original_pts.shape
    m = query_pts.shape[-1]
    ptsT = jnp.transpose(original_pts, (0, 2, 1))
    qT = jnp.transpose(query_pts, (0, 2, 1))

    cat, g = pl.pallas_call(
        _feat_body,
        grid=(b, n // _TN),
        in_specs=[
            pl.BlockSpec((1, _TN, 3), lambda bi, ni: (bi, ni, 0)),
            _full((3, _C)), _full((1, _C)),
            _full((_C, _C)), _full((1, _C)),
            _full((_C, _C)), _full((1, _C)),
            _full((_C, _C)), _full((1, _C)),
        ],
        out_specs=[
            pl.BlockSpec((1, _TN, _ROW), lambda bi, ni: (bi, ni, 0)),
            pl.BlockSpec((1, 1, _C), lambda bi, ni: (bi, 0, 0)),
        ],
        out_shape=[
            jax.ShapeDtypeStruct((b, n, _ROW), jnp.float32),
            jax.ShapeDtypeStruct((b, 1, _C), jnp.float32),
        ],
    )(ptsT, W_in, b_in.reshape(1, -1), W_b0, b_b0.reshape(1, -1),
      W_b1, b_b1.reshape(1, -1), W_b2, b_b2.reshape(1, -1))

    table = cat.reshape(b * n, _ROW)
    sc_gather = _make_sc_gather()

    outs = []
    for bi in range(b):
        idx = pl.pallas_call(
            functools.partial(_knn_body, row_offset=bi * n),
            grid=(m // _TM,),
            in_specs=[
                pl.BlockSpec((1, _TM, 3), lambda mi, bi=bi: (bi, mi, 0)),
                pl.BlockSpec((1, 3, n), lambda mi, bi=bi: (bi, 0, 0)),
            ],
            out_specs=pl.BlockSpec((1, _K, _TM, 1), lambda mi: (0, 0, mi, 0)),
            out_shape=jax.ShapeDtypeStruct((1, _K, m, 1), jnp.int32),
        )(qT, original_pts)

        G = sc_gather(table, idx.reshape(_ROWS_B))
        G = G.reshape(_K, m, _ROW)

        out_b = pl.pallas_call(
            _reg_body,
            grid=(m // _TM,),
            in_specs=[
                pl.BlockSpec((1, _TM, 3), lambda mi, bi=bi: (bi, mi, 0)),
                pl.BlockSpec((_K, _TM, _ROW), lambda mi: (0, mi, 0)),
                pl.BlockSpec((1, 1, _C), lambda mi, bi=bi: (bi, 0, 0)),
                _full((3 + _CAT + _C, 256)), _full((1, 256)),
                _full((256, 64)), _full((1, 64)),
                _full((64, 1)), _full((1, 1)),
            ],
            out_specs=pl.BlockSpec((1, _TM, 1), lambda mi: (0, mi, 0)),
            out_shape=jax.ShapeDtypeStruct((1, m, 1), jnp.float32),
        )(qT, G, g, R1, Rb1.reshape(1, -1),
          R2, Rb2.reshape(1, -1), R3, Rb3.reshape(1, -1))
        outs.append(out_b)

    out = jnp.concatenate(outs, axis=0)
    return jnp.transpose(out, (0, 2, 1))
